# Initial kernel scaffold; baseline (speedup 1.0000x reference)
import functools

import jax
import jax.numpy as jnp
from jax import lax
from jax.experimental import pallas as pl
from jax.experimental.pallas import tpu as pltpu
from jax.experimental.pallas import tpu_sc as plsc

NUM_CORES = 2
NUM_SUBCORES = 16
NUM_WORKERS = NUM_CORES * NUM_SUBCORES

CHUNK = 128


def _build_gather(total_b: int, embed_dim: int):
    assert total_b % (NUM_WORKERS * CHUNK) == 0
    b_per_w = total_b // NUM_WORKERS
    n_steps = b_per_w // CHUNK
    mesh = plsc.VectorSubcoreMesh(
        core_axis_name="c", subcore_axis_name="s",
        num_cores=NUM_CORES, num_subcores=NUM_SUBCORES)

    @functools.partial(
        pl.kernel,
        out_type=jax.ShapeDtypeStruct((total_b, embed_dim), jnp.float32),
        mesh=mesh,
        scratch_types=[
            pltpu.VMEM((CHUNK,), jnp.int32),
            pltpu.VMEM((CHUNK, embed_dim), jnp.float32),
            pltpu.SemaphoreType.DMA,
        ],
    )
    def gather_kernel(x_hbm, table_hbm, out_hbm, idx_v, rows_v, sem):
        wid = lax.axis_index("s") * NUM_CORES + lax.axis_index("c")
        base = wid * b_per_w

        def step(i, carry):
            off = base + i * CHUNK
            pltpu.sync_copy(x_hbm.at[pl.ds(off, CHUNK)], idx_v)
            pltpu.async_copy(table_hbm.at[idx_v], rows_v, sem).wait()
            pltpu.sync_copy(rows_v, out_hbm.at[pl.ds(off, CHUNK)])
            return carry

        lax.fori_loop(0, n_steps, step, 0)

    return gather_kernel


@jax.jit
def kernel(x, table):
    batch, hist = x.shape
    _, embed_dim = table.shape
    flat_idx = x.reshape(-1).astype(jnp.int32)
    out = _build_gather(batch * hist, embed_dim)(flat_idx, table)
    return out.reshape(batch, hist, embed_dim)


# speedup vs baseline: 1.5729x; 1.5729x over previous
import functools

import jax
import jax.numpy as jnp
from jax import lax
from jax.experimental import pallas as pl
from jax.experimental.pallas import tpu as pltpu
from jax.experimental.pallas import tpu_sc as plsc

NUM_CORES = 2
NUM_SUBCORES = 16
NUM_WORKERS = NUM_CORES * NUM_SUBCORES

CHUNK = 128


def _build_gather(total_b: int, embed_dim: int):
    assert total_b % (NUM_WORKERS * CHUNK) == 0
    b_per_w = total_b // NUM_WORKERS
    n_steps = b_per_w // CHUNK
    mesh = plsc.VectorSubcoreMesh(
        core_axis_name="c", subcore_axis_name="s",
        num_cores=NUM_CORES, num_subcores=NUM_SUBCORES)

    @functools.partial(
        pl.kernel,
        out_type=jax.ShapeDtypeStruct((total_b, embed_dim), jnp.float32),
        mesh=mesh,
        scratch_types=[
            pltpu.VMEM((CHUNK,), jnp.int32),
            pltpu.VMEM((CHUNK, embed_dim), jnp.float32),
            pltpu.SemaphoreType.DMA,
        ],
        compiler_params=pltpu.CompilerParams(use_tc_tiling_on_sc=False),
    )
    def gather_kernel(x_hbm, table_hbm, out_hbm, idx_v, rows_v, sem):
        wid = lax.axis_index("s") * NUM_CORES + lax.axis_index("c")
        base = wid * b_per_w

        def step(i, carry):
            off = base + i * CHUNK
            pltpu.sync_copy(x_hbm.at[pl.ds(off, CHUNK)], idx_v)
            pltpu.async_copy(table_hbm.at[idx_v], rows_v, sem).wait()
            pltpu.sync_copy(rows_v, out_hbm.at[pl.ds(off, CHUNK)])
            return carry

        lax.fori_loop(0, n_steps, step, 0)

    return gather_kernel


@jax.jit
def kernel(x, table):
    batch, hist = x.shape
    _, embed_dim = table.shape
    flat_idx = x.reshape(-1).astype(jnp.int32)
    out = _build_gather(batch * hist, embed_dim)(flat_idx, table)
    return out.reshape(batch, hist, embed_dim)


# speedup vs baseline: 1.8750x; 1.1921x over previous
import functools

import jax
import jax.numpy as jnp
from jax import lax
from jax.experimental import pallas as pl
from jax.experimental.pallas import tpu as pltpu
from jax.experimental.pallas import tpu_sc as plsc

NUM_CORES = 2
NUM_SUBCORES = 16
NUM_WORKERS = NUM_CORES * NUM_SUBCORES

SUB = 128
SUBS_PER_CHUNK = 5
CHUNK = SUB * SUBS_PER_CHUNK


def _build_gather(total_b: int, embed_dim: int):
    assert total_b % (NUM_WORKERS * CHUNK) == 0
    b_per_w = total_b // NUM_WORKERS
    n_steps = b_per_w // CHUNK
    assert n_steps % 2 == 0
    n_outer = n_steps // 2
    mesh = plsc.VectorSubcoreMesh(
        core_axis_name="c", subcore_axis_name="s",
        num_cores=NUM_CORES, num_subcores=NUM_SUBCORES)

    @functools.partial(
        pl.kernel,
        out_type=jax.ShapeDtypeStruct((total_b, embed_dim), jnp.float32),
        mesh=mesh,
        scratch_types=[
            pltpu.VMEM((b_per_w,), jnp.int32),
            pltpu.VMEM((CHUNK, embed_dim), jnp.float32),
            pltpu.VMEM((CHUNK, embed_dim), jnp.float32),
            pltpu.SemaphoreType.DMA,
            pltpu.SemaphoreType.DMA,
            pltpu.SemaphoreType.DMA,
            pltpu.SemaphoreType.DMA,
        ],
        compiler_params=pltpu.CompilerParams(use_tc_tiling_on_sc=False),
    )
    def gather_kernel(x_hbm, table_hbm, out_hbm, idx_v, rows0, rows1,
                      g_sem0, g_sem1, o_sem0, o_sem1):
        wid = lax.axis_index("s") * NUM_CORES + lax.axis_index("c")
        base = wid * b_per_w
        rows = (rows0, rows1)
        g_sems = (g_sem0, g_sem1)
        o_sems = (o_sem0, o_sem1)

        pltpu.sync_copy(x_hbm.at[pl.ds(base, b_per_w)], idx_v)

        def fire(chunk_i, buf, sem):
            for k in range(SUBS_PER_CHUNK):
                pltpu.async_copy(
                    table_hbm.at[idx_v.at[pl.ds(chunk_i * CHUNK + k * SUB, SUB)]],
                    buf.at[pl.ds(k * SUB, SUB)], sem)

        def drain_gathers(buf, sem):
            pltpu.make_async_copy(table_hbm.at[pl.ds(0, CHUNK)], buf, sem).wait()

        def wait_store(buf, sem):
            pltpu.make_async_copy(buf, out_hbm.at[pl.ds(0, CHUNK)], sem).wait()

        def store(chunk_i, buf, sem):
            pltpu.async_copy(buf, out_hbm.at[pl.ds(base + chunk_i * CHUNK, CHUNK)],
                             sem)

        fire(0, rows[0], g_sems[0])

        def outer(g, carry):
            for b in (0, 1):
                i = 2 * g + b
                if b == 0:
                    @pl.when(g > 0)
                    def _():
                        wait_store(rows[1], o_sems[1])
                else:
                    wait_store(rows[0], o_sems[0])
                if b == 0:
                    fire(i + 1, rows[1], g_sems[1])
                else:
                    @pl.when(g < n_outer - 1)
                    def _():
                        fire(i + 1, rows[0], g_sems[0])
                drain_gathers(rows[b], g_sems[b])
                store(i, rows[b], o_sems[b])
            return carry

        lax.fori_loop(0, n_outer, outer, 0)
        wait_store(rows[1], o_sems[1])

    return gather_kernel


@jax.jit
def kernel(x, table):
    batch, hist = x.shape
    _, embed_dim = table.shape
    flat_idx = x.reshape(-1).astype(jnp.int32)
    out = _build_gather(batch * hist, embed_dim)(flat_idx, table)
    return out.reshape(batch, hist, embed_dim)
